# baseline (speedup 1.0000x reference)
import functools

import jax
import jax.numpy as jnp
from jax import lax
from jax.experimental import pallas as pl
from jax.experimental.pallas import tpu as pltpu
from jax.experimental.pallas import tpu_sc as plsc

_V = 100000
_BB = 16


def _s_body(idx_hbm, prow_hbm, w1_hbm, table_hbm, s_hbm,
            idx_v, prow_v, w1_v, table_v, out_v):
    @pl.when((lax.axis_index("c") == 0) & (lax.axis_index("s") == 0))
    def _():
        pltpu.sync_copy(idx_hbm, idx_v)
        pltpu.sync_copy(prow_hbm, prow_v)
        pltpu.sync_copy(w1_hbm, w1_v)
        pltpu.sync_copy(table_hbm, table_v)
        acc = jnp.zeros((16,), jnp.float32)
        for h in range(2):
            iv = idx_v[pl.ds(h * 16, 16)]
            vals = plsc.load_gather(table_v, [iv])
            acc = acc + vals * prow_v[pl.ds(h * 16, 16)]
        tot = jnp.sum(acc)
        out_v[...] = tot * w1_v[...]
        pltpu.sync_copy(out_v, s_hbm)


def _s_sc(idx, prow, w1b, table):
    mesh = plsc.VectorSubcoreMesh(core_axis_name="c", subcore_axis_name="s")
    fn = pl.kernel(
        _s_body,
        out_type=jax.ShapeDtypeStruct((16,), jnp.float32),
        mesh=mesh,
        scratch_types=[
            pltpu.VMEM((32,), jnp.int32),
            pltpu.VMEM((32,), jnp.float32),
            pltpu.VMEM((16,), jnp.float32),
            pltpu.VMEM((_V,), jnp.float32),
            pltpu.VMEM((16,), jnp.float32),
        ],
        compiler_params=pltpu.CompilerParams(needs_layout_passes=False),
    )
    return fn(idx, prow, w1b, table)


def _main_body(s_ref, x_ref, f_ref, o_ref):
    s = s_ref[0]
    y = x_ref[...] + s * f_ref[...]
    m = jnp.max(y, axis=1, keepdims=True)
    l = jnp.log(jnp.sum(jnp.exp(y - m), axis=1, keepdims=True))
    o_ref[...] = y - m - l


def _main(s, x, f):
    batch, vocab = x.shape
    return pl.pallas_call(
        _main_body,
        grid=(batch // _BB,),
        in_specs=[
            pl.BlockSpec(memory_space=pltpu.SMEM),
            pl.BlockSpec((_BB, vocab), lambda b: (b, 0)),
            pl.BlockSpec((1, vocab), lambda b: (0, 0)),
        ],
        out_specs=pl.BlockSpec((_BB, vocab), lambda b: (b, 0)),
        out_shape=jax.ShapeDtypeStruct((batch, vocab), jnp.float32),
        compiler_params=pltpu.CompilerParams(
            dimension_semantics=("arbitrary",),
        ),
    )(s, x, f)


def kernel(decoder_output, decoded_words, pattern, w1, lv_table, i, j, batch_size):
    n = jnp.asarray(i, dtype=jnp.int32) * 7 + jnp.asarray(j, dtype=jnp.int32)
    cond = (n > 0) & (jnp.asarray(j) < 7) & (jnp.asarray(i) < 4)

    nd = decoded_words.shape[1]
    idx = jnp.pad(decoded_words[0], (0, 32 - nd))
    prow = jnp.pad(jnp.take(pattern, n, axis=0), (0, 4))
    w1b = jnp.broadcast_to(jnp.where(cond, w1[0], 0.0), (16,)).astype(jnp.float32)
    table = lv_table[:, 0]

    s16 = _s_sc(idx, prow, w1b, table)
    s = s16[:1]

    f = lv_table.reshape(1, -1)
    return _main(s, decoder_output, f)
